# baseline (speedup 1.0000x reference)
import functools

import jax
import jax.numpy as jnp
from jax import lax
from jax.experimental import pallas as pl
from jax.experimental.pallas import tpu as pltpu
from jax.experimental.pallas import tpu_sc as plsc

_N_PROT = 16
_MAX_LEN = 2048
_N_SPECIES = 64
_N_AAS = 25
_POS = _N_PROT * _MAX_LEN
_NW = 32
_P_PER_W = _POS // _NW
_L = 16
_BLOCKS = _P_PER_W // _L
_ACC = _P_PER_W * _N_AAS


def _pwm_body(ali_hbm, w_hbm, out_hbm, ali_v, out_v, w_v, w_rep, red_v,
              in_sem):
    wid = lax.axis_index("sub") * 2 + lax.axis_index("core")
    pos0 = wid * _P_PER_W

    cp_in = pltpu.async_copy(
        ali_hbm.at[pl.ds(pos0 * _N_SPECIES, _P_PER_W * _N_SPECIES)],
        ali_v, in_sem)
    pltpu.sync_copy(w_hbm, w_v)

    lane16 = lax.iota(jnp.int32, _L)

    def _allreduce(v, op):
        for k in (8, 4, 2, 1):
            red_v[...] = v
            v = op(v, plsc.load_gather(red_v, [lane16 ^ k]))
        return v

    wg = [w_v[pl.ds(g * _L, _L)] for g in range(4)]
    m = jnp.maximum(jnp.maximum(wg[0], wg[1]), jnp.maximum(wg[2], wg[3]))
    mx = _allreduce(m, jnp.maximum)
    eg = [jnp.exp(w - mx) for w in wg]
    tot = eg[0] + eg[1] + eg[2] + eg[3]
    inv = 1.0 / _allreduce(tot, jnp.add)
    for g in range(4):
        w_v[pl.ds(g * _L, _L)] = eg[g] * inv

    for s in range(_N_SPECIES):
        w_rep[pl.ds(s * _L, _L)] = plsc.load_gather(
            w_v, [jnp.full((_L,), s, jnp.int32)])

    zero = jnp.zeros((_L,), jnp.float32)

    def _zero(i, c):
        out_v[pl.ds(i * _L, _L)] = zero
        return c

    lax.fori_loop(0, _ACC // _L, _zero, 0)

    cp_in.wait()

    lane = lax.iota(jnp.int32, _L)
    in_step = lane * _N_SPECIES
    out_step = lane * _N_AAS

    def _block(b, c):
        base_in = in_step + b * (_L * _N_SPECIES)
        base_out = out_step + b * (_L * _N_AAS)
        for s in range(_N_SPECIES):
            col = plsc.load_gather(ali_v, [base_in + s])
            plsc.addupdate_scatter(
                out_v, [base_out + col], w_rep[pl.ds(s * _L, _L)])
        return c

    lax.fori_loop(0, _BLOCKS, _block, 0)

    pltpu.sync_copy(out_v, out_hbm.at[pl.ds(pos0 * _N_AAS, _ACC)])


_pwm = functools.partial(
    pl.kernel,
    out_type=jax.ShapeDtypeStruct((_POS * _N_AAS,), jnp.float32),
    mesh=plsc.VectorSubcoreMesh(core_axis_name="core", subcore_axis_name="sub"),
    compiler_params=pltpu.CompilerParams(needs_layout_passes=False),
    scratch_types=[
        pltpu.VMEM((_P_PER_W * _N_SPECIES,), jnp.int32),
        pltpu.VMEM((_ACC,), jnp.float32),
        pltpu.VMEM((_N_SPECIES,), jnp.float32),
        pltpu.VMEM((_N_SPECIES * _L,), jnp.float32),
        pltpu.VMEM((_L,), jnp.float32),
        pltpu.SemaphoreType.DMA,
    ],
)(_pwm_body)


@jax.jit
def kernel(alis0based, weights):
    ali_flat = alis0based.astype(jnp.int32).reshape((-1,))
    out = _pwm(ali_flat, weights.astype(jnp.float32))
    return out.reshape((_N_PROT, _MAX_LEN, _N_AAS))


# speedup vs baseline: 1.2502x; 1.2502x over previous
import functools

import jax
import jax.numpy as jnp
from jax import lax
from jax.experimental import pallas as pl
from jax.experimental.pallas import tpu as pltpu
from jax.experimental.pallas import tpu_sc as plsc

_N_PROT = 16
_MAX_LEN = 2048
_N_SPECIES = 64
_N_AAS = 25
_POS = _N_PROT * _MAX_LEN
_NW = 32
_P_PER_W = _POS // _NW
_L = 16
_BLOCKS = _P_PER_W // _L
_ACC = _P_PER_W * _N_AAS


def _pwm_body(ali_hbm, w_hbm, out_hbm, ali_v, out_v, w_v, w_rep, red_v,
              in_sem):
    wid = lax.axis_index("sub") * 2 + lax.axis_index("core")
    pos0 = wid * _P_PER_W

    cp_in = pltpu.async_copy(
        ali_hbm.at[pl.ds(pos0 * _N_SPECIES, _P_PER_W * _N_SPECIES)],
        ali_v, in_sem)
    pltpu.sync_copy(w_hbm, w_v)

    lane16 = lax.iota(jnp.int32, _L)

    def _allreduce(v, op):
        for k in (8, 4, 2, 1):
            red_v[...] = v
            v = op(v, plsc.load_gather(red_v, [lane16 ^ k]))
        return v

    wg = [w_v[pl.ds(g * _L, _L)] for g in range(4)]
    m = jnp.maximum(jnp.maximum(wg[0], wg[1]), jnp.maximum(wg[2], wg[3]))
    mx = _allreduce(m, jnp.maximum)
    eg = [jnp.exp(w - mx) for w in wg]
    tot = eg[0] + eg[1] + eg[2] + eg[3]
    inv = 1.0 / _allreduce(tot, jnp.add)
    for g in range(4):
        w_v[pl.ds(g * _L, _L)] = eg[g] * inv

    for s in range(_N_SPECIES):
        w_rep[pl.ds(s * _L, _L)] = plsc.load_gather(
            w_v, [jnp.full((_L,), s, jnp.int32)])

    zero = jnp.zeros((_L,), jnp.float32)
    _ZU = 8

    @plsc.parallel_loop(0, _ACC // _L, step=_ZU, unroll=2)
    def _zero(i):
        for u in range(_ZU):
            out_v[pl.ds((i + u) * _L, _L)] = zero

    cp_in.wait()

    in_step = lane16 * _N_SPECIES
    out_step = lane16 * _N_AAS

    def _species(s, c):
        w_s = w_rep[pl.ds(s * _L, _L)]

        @plsc.parallel_loop(0, _BLOCKS, unroll=8)
        def _blk(b):
            col = plsc.load_gather(ali_v, [in_step + (b * (_L * _N_SPECIES) + s)])
            plsc.addupdate_scatter(
                out_v, [(out_step + b * (_L * _N_AAS)) + col], w_s)

        return c

    lax.fori_loop(0, _N_SPECIES, _species, 0)

    pltpu.sync_copy(out_v, out_hbm.at[pl.ds(pos0 * _N_AAS, _ACC)])


_pwm = functools.partial(
    pl.kernel,
    out_type=jax.ShapeDtypeStruct((_POS * _N_AAS,), jnp.float32),
    mesh=plsc.VectorSubcoreMesh(core_axis_name="core", subcore_axis_name="sub"),
    compiler_params=pltpu.CompilerParams(needs_layout_passes=False),
    scratch_types=[
        pltpu.VMEM((_P_PER_W * _N_SPECIES,), jnp.int32),
        pltpu.VMEM((_ACC,), jnp.float32),
        pltpu.VMEM((_N_SPECIES,), jnp.float32),
        pltpu.VMEM((_N_SPECIES * _L,), jnp.float32),
        pltpu.VMEM((_L,), jnp.float32),
        pltpu.SemaphoreType.DMA,
    ],
)(_pwm_body)


@jax.jit
def kernel(alis0based, weights):
    ali_flat = alis0based.astype(jnp.int32).reshape((-1,))
    out = _pwm(ali_flat, weights.astype(jnp.float32))
    return out.reshape((_N_PROT, _MAX_LEN, _N_AAS))
